# Initial kernel scaffold; baseline (speedup 1.0000x reference)
import jax
import jax.numpy as jnp
from jax.experimental import pallas as pl


def kernel(x, t, W, b, Wg, bg):
    raise NotImplementedError("write your pallas kernel here")


# speedup vs baseline: 2.2255x; 2.2255x over previous
import jax
import jax.numpy as jnp
from jax.experimental import pallas as pl
from jax.experimental.pallas import tpu as pltpu

NUM_EXPERTS = 8
IN_DIM = 768
OUT_DIM = 768
B = 4096
T_LEN = 8
BLK = 512


def _fused_body(x_ref, t_ref, W_ref, b_ref, Wg_ref, bg_ref, o_ref):
    tm = jnp.mean(t_ref[...], axis=1)
    logits = jax.lax.dot_general(
        tm, Wg_ref[...], (((1,), (1,)), ((), ())),
        preferred_element_type=jnp.float32) + bg_ref[...]
    mx = jnp.max(logits, axis=1, keepdims=True)
    eids = jax.lax.broadcasted_iota(jnp.int32, logits.shape, 1)
    top1 = jnp.min(jnp.where(logits >= mx, eids, NUM_EXPERTS), axis=1)

    x = x_ref[...]
    acc = jnp.zeros((BLK, OUT_DIM), jnp.float32)
    for e in range(NUM_EXPERTS):
        ye = jax.lax.dot_general(
            x, W_ref[e], (((1,), (1,)), ((), ())),
            preferred_element_type=jnp.float32) + b_ref[e][None, :]
        acc = jnp.where((top1 == e)[:, None], ye, acc)
    o_ref[...] = acc


def kernel(x, t, W, b, Wg, bg):
    x_flat = jnp.squeeze(x, axis=1)
    bg2 = bg.reshape(1, NUM_EXPERTS)
    grid = (B // BLK,)
    out = pl.pallas_call(
        _fused_body,
        grid=grid,
        in_specs=[
            pl.BlockSpec((BLK, IN_DIM), lambda i: (i, 0)),
            pl.BlockSpec((BLK, T_LEN, OUT_DIM), lambda i: (i, 0, 0)),
            pl.BlockSpec((NUM_EXPERTS, OUT_DIM, IN_DIM), lambda i: (0, 0, 0)),
            pl.BlockSpec((NUM_EXPERTS, OUT_DIM), lambda i: (0, 0)),
            pl.BlockSpec((NUM_EXPERTS, OUT_DIM), lambda i: (0, 0)),
            pl.BlockSpec((1, NUM_EXPERTS), lambda i: (0, 0)),
        ],
        out_specs=pl.BlockSpec((BLK, OUT_DIM), lambda i: (i, 0)),
        out_shape=jax.ShapeDtypeStruct((B, OUT_DIM), jnp.float32),
    )(x_flat, t, W, b, Wg, bg2)
    return out.reshape(B, 1, OUT_DIM)
